# baseline (speedup 1.0000x reference)
import functools

import jax
import jax.numpy as jnp
from jax.experimental import pallas as pl

NA = 3
NO = 85


PAD = 128


def _head_kernel(x_ref, wt_ref, b_ref, out_ref):
    z = jax.lax.dot_general(
        x_ref[0].astype(jnp.bfloat16), wt_ref[...],
        dimension_numbers=(((0,), (0,)), ((), ())),
        preferred_element_type=jnp.float32,
    )
    z = z + b_ref[0]
    for a in range(NA):
        out_ref[0, a] = z[:, a * PAD:a * PAD + NO]


@functools.partial(jax.jit, static_argnames=("tile",))
def _head(x, W, b, tile):
    bs, c, ny, nx = x.shape
    hw = ny * nx
    xr = x.reshape(bs, c, hw)
    wt = jnp.zeros((c, NA * PAD), jnp.bfloat16)
    br = jnp.zeros((1, NA * PAD), jnp.float32)
    for a in range(NA):
        wt = wt.at[:, a * PAD:a * PAD + NO].set(
            W[a * NO:(a + 1) * NO].T.astype(jnp.bfloat16))
        br = br.at[0, a * PAD:a * PAD + NO].set(b[a * NO:(a + 1) * NO])
    grid = (bs, hw // tile)
    out = pl.pallas_call(
        _head_kernel,
        grid=grid,
        in_specs=[
            pl.BlockSpec((1, c, tile), lambda i, j: (i, 0, j)),
            pl.BlockSpec((c, NA * PAD), lambda i, j: (0, 0)),
            pl.BlockSpec((1, NA * PAD), lambda i, j: (0, 0)),
        ],
        out_specs=pl.BlockSpec((1, NA, tile, NO), lambda i, j: (i, 0, j, 0)),
        out_shape=jax.ShapeDtypeStruct((bs, NA, hw, NO), jnp.float32),
    )(xr, wt, br)
    return out.reshape(bs, NA, ny, nx, NO)


def kernel(x0, x1, x2, W0, b0, W1, b1, W2, b2):
    y0 = _head(x0, W0, b0, tile=512)
    y1 = _head(x1, W1, b1, tile=512)
    y2 = _head(x2, W2, b2, tile=256)
    return (y0, y1, y2)


# speedup vs baseline: 1.2970x; 1.2970x over previous
import functools

import jax
import jax.numpy as jnp
from jax.experimental import pallas as pl

NA = 3
NO = 85


PAD = 128


def _head_kernel(x_ref, wt_ref, b_ref, out_ref):
    z = jax.lax.dot_general(
        x_ref[0].astype(jnp.bfloat16), wt_ref[...],
        dimension_numbers=(((0,), (0,)), ((), ())),
        preferred_element_type=jnp.float32,
    )
    z = z + b_ref[0]
    for a in range(NA):
        out_ref[0, a] = z[:, a * PAD:a * PAD + NO]


@functools.partial(jax.jit, static_argnames=("tile",))
def _head(x, W, b, tile):
    bs, c, ny, nx = x.shape
    hw = ny * nx
    xr = x.reshape(bs, c, hw)
    wt = jnp.zeros((c, NA * PAD), jnp.bfloat16)
    br = jnp.zeros((1, NA * PAD), jnp.float32)
    for a in range(NA):
        wt = wt.at[:, a * PAD:a * PAD + NO].set(
            W[a * NO:(a + 1) * NO].T.astype(jnp.bfloat16))
        br = br.at[0, a * PAD:a * PAD + NO].set(b[a * NO:(a + 1) * NO])
    grid = (bs, hw // tile)
    out = pl.pallas_call(
        _head_kernel,
        grid=grid,
        in_specs=[
            pl.BlockSpec((1, c, tile), lambda i, j: (i, 0, j)),
            pl.BlockSpec((c, NA * PAD), lambda i, j: (0, 0)),
            pl.BlockSpec((1, NA * PAD), lambda i, j: (0, 0)),
        ],
        out_specs=pl.BlockSpec((1, NA, tile, NO), lambda i, j: (i, 0, j, 0)),
        out_shape=jax.ShapeDtypeStruct((bs, NA, hw, NO), jnp.float32),
    )(xr, wt, br)
    return out.reshape(bs, NA, ny, nx, NO)


def kernel(x0, x1, x2, W0, b0, W1, b1, W2, b2):
    y0 = _head(x0, W0, b0, tile=4096)
    y1 = _head(x1, W1, b1, tile=1024)
    y2 = _head(x2, W2, b2, tile=256)
    return (y0, y1, y2)


# speedup vs baseline: 1.4882x; 1.1474x over previous
import jax
import jax.numpy as jnp
from jax.experimental import pallas as pl

NA = 3
NO = 85
PAD = 128


def _detect_kernel(x0_ref, x1_ref, x2_ref,
                   wt0_ref, wt1_ref, wt2_ref, b_ref,
                   out0_ref, out1_ref, out2_ref):
    for x_ref, wt_ref, lvl, out_ref in (
            (x0_ref, wt0_ref, 0, out0_ref),
            (x1_ref, wt1_ref, 1, out1_ref),
            (x2_ref, wt2_ref, 2, out2_ref)):
        z = jax.lax.dot_general(
            x_ref[0].astype(jnp.bfloat16), wt_ref[...],
            dimension_numbers=(((0,), (0,)), ((), ())),
            preferred_element_type=jnp.float32,
        )
        z = z + b_ref[lvl]
        for a in range(NA):
            out_ref[0, a] = z[:, a * PAD:a * PAD + NO]


def _pack_w(W, b):
    c = W.shape[1]
    wt = jnp.zeros((c, NA * PAD), jnp.bfloat16)
    br = jnp.zeros((NA * PAD,), jnp.float32)
    for a in range(NA):
        wt = wt.at[:, a * PAD:a * PAD + NO].set(
            W[a * NO:(a + 1) * NO].T.astype(jnp.bfloat16))
        br = br.at[a * PAD:a * PAD + NO].set(b[a * NO:(a + 1) * NO])
    return wt, br


@jax.jit
def _detect(x0, x1, x2, W0, b0, W1, b1, W2, b2):
    bs = x0.shape[0]
    shapes = [x.shape for x in (x0, x1, x2)]
    xr = [x.reshape(x.shape[0], x.shape[1], -1) for x in (x0, x1, x2)]
    packed = [_pack_w(W, b) for W, b in ((W0, b0), (W1, b1), (W2, b2))]
    wts = [p[0] for p in packed]
    brs = jnp.stack([p[1] for p in packed])

    def x_spec(c, hw):
        return pl.BlockSpec((1, c, hw), lambda i: (i, 0, 0))

    def w_spec(c):
        return pl.BlockSpec((c, NA * PAD), lambda i: (0, 0))

    def o_spec(hw):
        return pl.BlockSpec((1, NA, hw, NO), lambda i: (i, 0, 0, 0))

    outs = pl.pallas_call(
        _detect_kernel,
        grid=(bs,),
        in_specs=(
            [x_spec(s[1], s[2] * s[3]) for s in shapes]
            + [w_spec(s[1]) for s in shapes]
            + [pl.BlockSpec((3, NA * PAD), lambda i: (0, 0))]
        ),
        out_specs=[o_spec(s[2] * s[3]) for s in shapes],
        out_shape=[
            jax.ShapeDtypeStruct((bs, NA, s[2] * s[3], NO), jnp.float32)
            for s in shapes],
    )(*xr, *wts, brs)
    return tuple(
        o.reshape(bs, NA, s[2], s[3], NO) for o, s in zip(outs, shapes))


def kernel(x0, x1, x2, W0, b0, W1, b1, W2, b2):
    return _detect(x0, x1, x2, W0, b0, W1, b1, W2, b2)


# speedup vs baseline: 1.5659x; 1.0522x over previous
import jax
import jax.numpy as jnp
from jax.experimental import pallas as pl

NA = 3
NO = 85
NP = 256


def _detect_kernel(x0_ref, x1_ref, x2_ref,
                   wt0_ref, wt1_ref, wt2_ref, b_ref,
                   out0_ref, out1_ref, out2_ref):
    for x_ref, wt_ref, lvl, out_ref in (
            (x0_ref, wt0_ref, 0, out0_ref),
            (x1_ref, wt1_ref, 1, out1_ref),
            (x2_ref, wt2_ref, 2, out2_ref)):
        z = jax.lax.dot_general(
            x_ref[0].astype(jnp.bfloat16), wt_ref[...],
            dimension_numbers=(((0,), (0,)), ((), ())),
            preferred_element_type=jnp.float32,
        )
        z = z + b_ref[lvl]
        for a in range(NA):
            out_ref[0, a] = z[:, a * NO:(a + 1) * NO]


def _pack_w(W, b):
    c = W.shape[1]
    wt = jnp.zeros((c, NP), jnp.bfloat16)
    wt = wt.at[:, :NA * NO].set(W.T.astype(jnp.bfloat16))
    br = jnp.zeros((NP,), jnp.float32).at[:NA * NO].set(b)
    return wt, br


@jax.jit
def _detect(x0, x1, x2, W0, b0, W1, b1, W2, b2):
    bs = x0.shape[0]
    shapes = [x.shape for x in (x0, x1, x2)]
    xr = [x.reshape(x.shape[0], x.shape[1], -1) for x in (x0, x1, x2)]
    packed = [_pack_w(W, b) for W, b in ((W0, b0), (W1, b1), (W2, b2))]
    wts = [p[0] for p in packed]
    brs = jnp.stack([p[1] for p in packed])

    def x_spec(c, hw):
        return pl.BlockSpec((1, c, hw), lambda i: (i, 0, 0))

    def w_spec(c):
        return pl.BlockSpec((c, NP), lambda i: (0, 0))

    def o_spec(hw):
        return pl.BlockSpec((1, NA, hw, NO), lambda i: (i, 0, 0, 0))

    outs = pl.pallas_call(
        _detect_kernel,
        grid=(bs,),
        in_specs=(
            [x_spec(s[1], s[2] * s[3]) for s in shapes]
            + [w_spec(s[1]) for s in shapes]
            + [pl.BlockSpec((3, NP), lambda i: (0, 0))]
        ),
        out_specs=[o_spec(s[2] * s[3]) for s in shapes],
        out_shape=[
            jax.ShapeDtypeStruct((bs, NA, s[2] * s[3], NO), jnp.float32)
            for s in shapes],
    )(*xr, *wts, brs)
    return tuple(
        o.reshape(bs, NA, s[2], s[3], NO) for o, s in zip(outs, shapes))


def kernel(x0, x1, x2, W0, b0, W1, b1, W2, b2):
    return _detect(x0, x1, x2, W0, b0, W1, b1, W2, b2)


# speedup vs baseline: 1.5677x; 1.0012x over previous
import jax
import jax.numpy as jnp
from jax.experimental import pallas as pl

NA = 3
NO = 85
NP = 256


def _detect_kernel(x0_ref, x1_ref, x2_ref,
                   wt0_ref, wt1_ref, wt2_ref, b_ref,
                   out0_ref, out1_ref, out2_ref):
    for x_ref, wt_ref, lvl, out_ref in (
            (x0_ref, wt0_ref, 0, out0_ref),
            (x1_ref, wt1_ref, 1, out1_ref),
            (x2_ref, wt2_ref, 2, out2_ref)):
        xb = x_ref[0].astype(jnp.bfloat16)
        for a in range(NA):
            z = jax.lax.dot_general(
                xb, wt_ref[a],
                dimension_numbers=(((0,), (0,)), ((), ())),
                preferred_element_type=jnp.float32,
            )
            out_ref[0, a] = z + b_ref[lvl, a]


def _pack_w(W, b):
    c = W.shape[1]
    wt = W.reshape(NA, NO, c).transpose(0, 2, 1).astype(jnp.bfloat16)
    br = b.reshape(NA, NO)
    return wt, br


@jax.jit
def _detect(x0, x1, x2, W0, b0, W1, b1, W2, b2):
    bs = x0.shape[0]
    shapes = [x.shape for x in (x0, x1, x2)]
    xr = [x.reshape(x.shape[0], x.shape[1], -1) for x in (x0, x1, x2)]
    packed = [_pack_w(W, b) for W, b in ((W0, b0), (W1, b1), (W2, b2))]
    wts = [p[0] for p in packed]
    brs = jnp.stack([p[1] for p in packed])

    def x_spec(c, hw):
        return pl.BlockSpec((1, c, hw), lambda i: (i, 0, 0))

    def w_spec(c):
        return pl.BlockSpec((NA, c, NO), lambda i: (0, 0, 0))

    def o_spec(hw):
        return pl.BlockSpec((1, NA, hw, NO), lambda i: (i, 0, 0, 0))

    outs = pl.pallas_call(
        _detect_kernel,
        grid=(bs,),
        in_specs=(
            [x_spec(s[1], s[2] * s[3]) for s in shapes]
            + [w_spec(s[1]) for s in shapes]
            + [pl.BlockSpec((3, NA, NO), lambda i: (0, 0, 0))]
        ),
        out_specs=[o_spec(s[2] * s[3]) for s in shapes],
        out_shape=[
            jax.ShapeDtypeStruct((bs, NA, s[2] * s[3], NO), jnp.float32)
            for s in shapes],
    )(*xr, *wts, brs)
    return tuple(
        o.reshape(bs, NA, s[2], s[3], NO) for o, s in zip(outs, shapes))


def kernel(x0, x1, x2, W0, b0, W1, b1, W2, b2):
    return _detect(x0, x1, x2, W0, b0, W1, b1, W2, b2)
